# Initial kernel scaffold; baseline (speedup 1.0000x reference)
import functools

import jax
import jax.numpy as jnp
from jax import lax
from jax.experimental import pallas as pl
from jax.experimental.pallas import tpu as pltpu
from jax.experimental.pallas import tpu_sc as plsc

N = 10000
E = 320000
H = 8
C = 16
D = H * C
T1W = D + 16
T2W = 32
AW = D + 16
WFW = D + 32

BLK = 1000
GRID = N // BLK

NC, NS = 2, 16
NW = NC * NS
EPW = E // NW
CH = 80
NCHUNK = EPW // CH
RPT = N // NS


def _dense_body(x_ref, w_ref, t1_ref, ad_ref, msrc_ref):
    blk = jnp.dot(x_ref[...], w_ref[...], preferred_element_type=jnp.float32)
    t1_ref[...] = blk[:, :T1W]
    ad_ref[...] = blk[:, T1W:WFW]
    red = jnp.max(blk[:, D:T1W], axis=0, keepdims=True)
    @pl.when(pl.program_id(0) == 0)
    def _():
        msrc_ref[...] = red
    @pl.when(pl.program_id(0) != 0)
    def _():
        msrc_ref[...] = jnp.maximum(msrc_ref[...], red)


def _t2_body(ad_ref, msrc_ref, t2_ref):
    adst = ad_ref[...]
    s = msrc_ref[...] + adst
    m = jnp.maximum(s, 0.2 * s)
    t2_ref[...] = jnp.concatenate([adst, m], axis=-1)


def _softmax_epilogue(a0_ref, a1_ref, erep_ref):
    u = a0_ref[...] + a1_ref[...]
    den = jnp.dot(u[:, D:D + H], erep_ref[...],
                  preferred_element_type=jnp.float32)
    return u[:, :D] / (den + 1e-16)


def _layer_norm(g, lng_ref, lnb_ref):
    m = jnp.mean(g, axis=-1, keepdims=True)
    v = jnp.mean((g - m) ** 2, axis=-1, keepdims=True)
    return (g - m) * lax.rsqrt(v + 1e-5) * lng_ref[...] + lnb_ref[...]


def _mid_body(a0_ref, a1_ref, res_ref, erep_ref, w_ref, lng_ref, lnb_ref,
              bias_ref, y_ref, t1_ref, ad_ref, msrc_ref):
    g = _softmax_epilogue(a0_ref, a1_ref, erep_ref) + bias_ref[...] + res_ref[...]
    y = jnp.maximum(_layer_norm(g, lng_ref, lnb_ref), 0.0)
    y_ref[...] = y
    blk = jnp.dot(y, w_ref[...], preferred_element_type=jnp.float32)
    t1_ref[...] = blk[:, :T1W]
    ad_ref[...] = blk[:, T1W:WFW]
    red = jnp.max(blk[:, D:T1W], axis=0, keepdims=True)
    @pl.when(pl.program_id(0) == 0)
    def _():
        msrc_ref[...] = red
    @pl.when(pl.program_id(0) != 0)
    def _():
        msrc_ref[...] = jnp.maximum(msrc_ref[...], red)


def _fin_body(a0_ref, a1_ref, erep_ref, sfold_ref, b2_ref, lng_ref, lnb_ref,
              linw_ref, linb_ref, out_ref):
    v = _softmax_epilogue(a0_ref, a1_ref, erep_ref)
    o16 = jnp.dot(v, sfold_ref[...], preferred_element_type=jnp.float32)
    o16 = o16 + b2_ref[...]
    y = jnp.maximum(_layer_norm(o16, lng_ref, lnb_ref), 0.0)
    out_ref[...] = jnp.dot(y, linw_ref[...],
                           preferred_element_type=jnp.float32) + linb_ref[...]


def _full(shape):
    return pl.BlockSpec(shape, lambda i: tuple(0 for _ in shape))


def _rows(width):
    return pl.BlockSpec((BLK, width), lambda i: (i, 0))


_dense_call = pl.pallas_call(
    _dense_body,
    grid=(GRID,),
    in_specs=[_rows(D), _full((D, WFW))],
    out_specs=[_rows(T1W), _rows(16), _full((1, 16))],
    out_shape=[
        jax.ShapeDtypeStruct((N, T1W), jnp.float32),
        jax.ShapeDtypeStruct((N, 16), jnp.float32),
        jax.ShapeDtypeStruct((1, 16), jnp.float32),
    ],
)

_t2_call = pl.pallas_call(
    _t2_body,
    grid=(GRID,),
    in_specs=[_rows(16), _full((1, 16))],
    out_specs=_rows(T2W),
    out_shape=jax.ShapeDtypeStruct((N, T2W), jnp.float32),
)

_mid_call = pl.pallas_call(
    _mid_body,
    grid=(GRID,),
    in_specs=[_rows(AW), _rows(AW), _rows(D), _full((H, D)), _full((D, WFW)),
              _full((1, D)), _full((1, D)), _full((1, D))],
    out_specs=[_rows(D), _rows(T1W), _rows(16), _full((1, 16))],
    out_shape=[
        jax.ShapeDtypeStruct((N, D), jnp.float32),
        jax.ShapeDtypeStruct((N, T1W), jnp.float32),
        jax.ShapeDtypeStruct((N, 16), jnp.float32),
        jax.ShapeDtypeStruct((1, 16), jnp.float32),
    ],
)

_fin_call = pl.pallas_call(
    _fin_body,
    grid=(GRID,),
    in_specs=[_rows(AW), _rows(AW), _full((H, D)), _full((D, C)),
              _full((1, C)), _full((1, C)), _full((1, C)), _full((C, 2)),
              _full((1, 2))],
    out_specs=_rows(2),
    out_shape=jax.ShapeDtypeStruct((N, 2), jnp.float32),
)


_sc_mesh = plsc.VectorSubcoreMesh(
    core_axis_name="c", subcore_axis_name="s", num_cores=NC, num_subcores=NS)


@functools.partial(
    pl.kernel,
    out_type=(jax.ShapeDtypeStruct((N, AW), jnp.float32),
              jax.ShapeDtypeStruct((N, AW), jnp.float32)),
    mesh=_sc_mesh,
    scratch_types=[
        pltpu.VMEM((CH,), jnp.int32),
        pltpu.VMEM((CH,), jnp.int32),
        pltpu.VMEM((CH, T1W), jnp.float32),
        pltpu.VMEM((CH, T2W), jnp.float32),
        pltpu.VMEM((CH, AW), jnp.float32),
        pltpu.VMEM((16,), jnp.float32),
        pltpu.VMEM_SHARED((N, AW), jnp.float32),
        pltpu.SemaphoreType.DMA,
        pltpu.SemaphoreType.DMA,
    ],
)
def _edge_call(t1_hbm, t2_hbm, src_hbm, dst_hbm, zeros_hbm,
               out0_hbm, out1_hbm,
               sidx, didx, buf1, buf2, msg, exv, acc, sem1, sem2):
    cid = lax.axis_index("c")
    sid = lax.axis_index("s")
    wid = sid * NC + cid
    rs = sid * RPT

    pltpu.sync_copy(zeros_hbm, acc.at[pl.ds(rs, RPT)])
    plsc.subcore_barrier()

    base_w = wid * EPW

    def chunk_body(t, carry):
        base = base_w + t * CH
        pltpu.sync_copy(src_hbm.at[pl.ds(base, CH)], sidx)
        pltpu.sync_copy(dst_hbm.at[pl.ds(base, CH)], didx)
        pltpu.async_copy(t1_hbm.at[sidx], buf1, sem1).wait()
        pltpu.async_copy(t2_hbm.at[didx], buf2, sem2).wait()

        def edge_body(j, carry2):
            asrc = buf1[j, pl.ds(D, 16)]
            advec = buf2[j, pl.ds(0, 16)]
            mvec = buf2[j, pl.ds(16, 16)]
            t0 = asrc + advec
            ex = jnp.exp(jnp.maximum(t0, 0.2 * t0) - mvec)
            exv[...] = ex
            msg[j, pl.ds(D, 16)] = ex
            for h in range(H):
                bex = plsc.load_gather(exv, [jnp.full((16,), h, jnp.int32)])
                msg[j, pl.ds(h * C, C)] = bex * buf1[j, pl.ds(h * C, C)]
            return carry2

        lax.fori_loop(0, CH, edge_body, 0)
        pltpu.sync_copy(msg, acc.at[didx], add=True)
        return carry

    lax.fori_loop(0, NCHUNK, chunk_body, 0)
    plsc.subcore_barrier()

    @pl.when(cid == 0)
    def _():
        pltpu.sync_copy(acc.at[pl.ds(rs, RPT)], out0_hbm.at[pl.ds(rs, RPT)])

    @pl.when(cid == 1)
    def _():
        pltpu.sync_copy(acc.at[pl.ds(rs, RPT)], out1_hbm.at[pl.ds(rs, RPT)])


def _amat(a):
    ar = a.reshape(H, C).astype(jnp.float32)
    eye = jnp.eye(H, dtype=jnp.float32)
    return (ar[:, :, None] * eye[:, None, :]).reshape(D, H)


def _wfull(W, a_s, a_d):
    As = _amat(a_s)
    Ad = _amat(a_d)
    return jnp.concatenate([W.astype(jnp.float32), As, As, Ad, Ad], axis=1)


def kernel(x, edge_index, W0, as0, ad0, b0, ln0_g, ln0_b,
           W1, as1, ad1, b1, ln1_g, ln1_b,
           W2, as2, ad2, b2, ln2_g, ln2_b, linW, linb):
    src = edge_index[0].astype(jnp.int32)
    dst = edge_index[1].astype(jnp.int32)

    erep = jnp.repeat(jnp.eye(H, dtype=jnp.float32), C, axis=1)
    sfold = jnp.tile(jnp.eye(C, dtype=jnp.float32), (H, 1)) / H
    zrows = jnp.zeros((RPT, AW), jnp.float32)

    row = lambda v: v.reshape(1, -1).astype(jnp.float32)

    t1, ad16, msrc = _dense_call(x, _wfull(W0, as0, ad0))
    t2 = _t2_call(ad16, msrc)
    a0, a1 = _edge_call(t1, t2, src, dst, zrows)

    y1, t1, ad16, msrc = _mid_call(a0, a1, x, erep, _wfull(W1, as1, ad1),
                                   row(ln0_g), row(ln0_b), row(b0))
    t2 = _t2_call(ad16, msrc)
    a0, a1 = _edge_call(t1, t2, src, dst, zrows)

    y2, t1, ad16, msrc = _mid_call(a0, a1, y1, erep, _wfull(W2, as2, ad2),
                                   row(ln1_g), row(ln1_b), row(b1))
    t2 = _t2_call(ad16, msrc)
    a0, a1 = _edge_call(t1, t2, src, dst, zrows)

    return _fin_call(a0, a1, erep, sfold, row(b2), row(ln2_g), row(ln2_b),
                     linW.astype(jnp.float32), row(linb))


# speedup vs baseline: 168.1093x; 168.1093x over previous
import functools

import jax
import jax.numpy as jnp
from jax import lax
from jax.experimental import pallas as pl
from jax.experimental.pallas import tpu as pltpu
from jax.experimental.pallas import tpu_sc as plsc

N = 10000
E = 320000
H = 8
C = 16
D = H * C
T1W = D + 16
AW = D + 16
WFW = D + 32

BLK = 1000
GRID = N // BLK

NC, NS = 2, 16
NW = NC * NS
EPW = E // NW
CH = 80
NCHUNK = EPW // CH
RPT = 624
RPT_LAST = N - (NS - 1) * RPT


def _dense_body(x_ref, w_ref, t1_ref, ad_ref, msrc_ref):
    blk = jnp.dot(x_ref[...], w_ref[...], preferred_element_type=jnp.float32)
    t1_ref[...] = blk[:, :T1W]
    ad_ref[...] = blk[:, T1W:WFW]
    red = jnp.max(blk[:, D:T1W], axis=0, keepdims=True)
    @pl.when(pl.program_id(0) == 0)
    def _():
        msrc_ref[...] = red
    @pl.when(pl.program_id(0) != 0)
    def _():
        msrc_ref[...] = jnp.maximum(msrc_ref[...], red)


def _softmax_epilogue(a0_ref, a1_ref, erep_ref):
    u = a0_ref[...] + a1_ref[...]
    den = jnp.dot(u[:, D:D + H], erep_ref[...],
                  preferred_element_type=jnp.float32)
    return u[:, :D] / (den + 1e-16)


def _layer_norm(g, lng_ref, lnb_ref):
    m = jnp.mean(g, axis=-1, keepdims=True)
    v = jnp.mean((g - m) ** 2, axis=-1, keepdims=True)
    return (g - m) * lax.rsqrt(v + 1e-5) * lng_ref[...] + lnb_ref[...]


def _mid_body(a0_ref, a1_ref, res_ref, erep_ref, w_ref, lng_ref, lnb_ref,
              bias_ref, y_ref, t1_ref, ad_ref, msrc_ref):
    g = _softmax_epilogue(a0_ref, a1_ref, erep_ref) + bias_ref[...] + res_ref[...]
    y = jnp.maximum(_layer_norm(g, lng_ref, lnb_ref), 0.0)
    y_ref[...] = y
    blk = jnp.dot(y, w_ref[...], preferred_element_type=jnp.float32)
    t1_ref[...] = blk[:, :T1W]
    ad_ref[...] = blk[:, T1W:WFW]
    red = jnp.max(blk[:, D:T1W], axis=0, keepdims=True)
    @pl.when(pl.program_id(0) == 0)
    def _():
        msrc_ref[...] = red
    @pl.when(pl.program_id(0) != 0)
    def _():
        msrc_ref[...] = jnp.maximum(msrc_ref[...], red)


def _fin_body(a0_ref, a1_ref, erep_ref, sfold_ref, b2_ref, lng_ref, lnb_ref,
              linw_ref, linb_ref, out_ref):
    v = _softmax_epilogue(a0_ref, a1_ref, erep_ref)
    o16 = jnp.dot(v, sfold_ref[...], preferred_element_type=jnp.float32)
    o16 = o16 + b2_ref[...]
    y = jnp.maximum(_layer_norm(o16, lng_ref, lnb_ref), 0.0)
    out_ref[...] = jnp.dot(y, linw_ref[...],
                           preferred_element_type=jnp.float32) + linb_ref[...]


def _full(shape):
    return pl.BlockSpec(shape, lambda i: tuple(0 for _ in shape))


def _rows(width):
    return pl.BlockSpec((BLK, width), lambda i: (i, 0))


_dense_call = pl.pallas_call(
    _dense_body,
    grid=(GRID,),
    in_specs=[_rows(D), _full((D, WFW))],
    out_specs=[_rows(T1W), _rows(16), _full((1, 16))],
    out_shape=[
        jax.ShapeDtypeStruct((N, T1W), jnp.float32),
        jax.ShapeDtypeStruct((N, 16), jnp.float32),
        jax.ShapeDtypeStruct((1, 16), jnp.float32),
    ],
)

_mid_call = pl.pallas_call(
    _mid_body,
    grid=(GRID,),
    in_specs=[_rows(AW), _rows(AW), _rows(D), _full((H, D)), _full((D, WFW)),
              _full((1, D)), _full((1, D)), _full((1, D))],
    out_specs=[_rows(D), _rows(T1W), _rows(16), _full((1, 16))],
    out_shape=[
        jax.ShapeDtypeStruct((N, D), jnp.float32),
        jax.ShapeDtypeStruct((N, T1W), jnp.float32),
        jax.ShapeDtypeStruct((N, 16), jnp.float32),
        jax.ShapeDtypeStruct((1, 16), jnp.float32),
    ],
)

_fin_call = pl.pallas_call(
    _fin_body,
    grid=(GRID,),
    in_specs=[_rows(AW), _rows(AW), _full((H, D)), _full((D, C)),
              _full((1, C)), _full((1, C)), _full((1, C)), _full((C, 2)),
              _full((1, 2))],
    out_specs=_rows(2),
    out_shape=jax.ShapeDtypeStruct((N, 2), jnp.float32),
)


_GATHER_DNUMS = lax.GatherDimensionNumbers(
    offset_dims=(), collapsed_slice_dims=(0,), start_index_map=(0,))


def _bcast_lane(vec, lane):
    idx = jnp.full((16, 1), lane, jnp.int32)
    return lax.gather(vec, idx, _GATHER_DNUMS, (1,),
                      mode=lax.GatherScatterMode.PROMISE_IN_BOUNDS)


_sc_mesh = plsc.VectorSubcoreMesh(
    core_axis_name="c", subcore_axis_name="s", num_cores=NC, num_subcores=NS)


@functools.partial(
    pl.kernel,
    out_type=(jax.ShapeDtypeStruct((N, AW), jnp.float32),
              jax.ShapeDtypeStruct((N, AW), jnp.float32)),
    mesh=_sc_mesh,
    scratch_types=[
        pltpu.VMEM((2, CH), jnp.int32),
        pltpu.VMEM((2, CH), jnp.int32),
        pltpu.VMEM((CH,), jnp.int32),
        pltpu.VMEM((2, CH, T1W), jnp.float32),
        pltpu.VMEM((2, CH, 16), jnp.float32),
        pltpu.VMEM((CH, AW), jnp.float32),
        pltpu.VMEM((1, 16), jnp.float32),
        pltpu.VMEM_SHARED((N, AW), jnp.float32),
        pltpu.SemaphoreType.DMA,
        pltpu.SemaphoreType.DMA,
        pltpu.SemaphoreType.DMA,
        pltpu.SemaphoreType.DMA,
    ],
    compiler_params=pltpu.CompilerParams(use_tc_tiling_on_sc=False),
)
def _edge_call(t1_hbm, t2_hbm, msrc_hbm, src_hbm, dst_hbm, zeros_hbm,
               out0_hbm, out1_hbm,
               sidx, didx, didx_sc, buf1, buf2, msg, msrcbuf, acc,
               isem, g1sem, g2sem, ssem):
    cid = lax.axis_index("c")
    sid = lax.axis_index("s")
    wid = sid * NC + cid
    rs = pl.multiple_of(sid * RPT, 8)

    pltpu.sync_copy(msrc_hbm, msrcbuf)
    msrc_v = msrcbuf[0, :]

    @pl.when(sid < NS - 1)
    def _():
        pltpu.sync_copy(zeros_hbm.at[pl.ds(0, RPT)], acc.at[pl.ds(rs, RPT)])

    @pl.when(sid == NS - 1)
    def _():
        pltpu.sync_copy(zeros_hbm, acc.at[pl.ds((NS - 1) * RPT, RPT_LAST)])

    plsc.subcore_barrier()

    base_w = wid * EPW

    def compute(b):
        @plsc.parallel_loop(0, CH, step=1, unroll=4)
        def edge_body(j):
            asrc = buf1[b, j, pl.ds(D, 16)]
            advec = buf2[b, j, pl.ds(0, 16)]
            t0 = asrc + advec
            s = msrc_v + advec
            mvec = jnp.maximum(s, 0.2 * s)
            ex = jnp.exp(jnp.maximum(t0, 0.2 * t0) - mvec)
            msg[j, pl.ds(D, 16)] = ex
            for h in range(H):
                bex = _bcast_lane(ex, h)
                msg[j, pl.ds(h * C, C)] = bex * buf1[b, j, pl.ds(h * C, C)]

    def didx_copy(b):
        for off in range(0, CH, 16):
            didx_sc[pl.ds(off, 16)] = didx[b, pl.ds(off, 16)]

    def scat_start():
        pltpu.async_copy(msg, acc.at[didx_sc], ssem, add=True)

    def scat_wait():
        pltpu.make_async_copy(msg, acc.at[didx_sc], ssem).wait()

    pltpu.sync_copy(src_hbm.at[pl.ds(base_w, CH)], sidx.at[0])
    pltpu.sync_copy(dst_hbm.at[pl.ds(base_w, CH)], didx.at[0])
    pltpu.async_copy(t1_hbm.at[sidx.at[0]], buf1.at[0], g1sem)
    pltpu.async_copy(t2_hbm.at[didx.at[0]], buf2.at[0], g2sem)
    pltpu.async_copy(src_hbm.at[pl.ds(base_w + CH, CH)], sidx.at[1], isem)
    pltpu.async_copy(dst_hbm.at[pl.ds(base_w + CH, CH)], didx.at[1], isem)

    def pair_body(p, carry):
        for b in range(2):
            cur = 2 * p + b
            nb = 1 - b
            pltpu.make_async_copy(t1_hbm.at[sidx.at[b]], buf1.at[b], g1sem).wait()
            pltpu.make_async_copy(t2_hbm.at[didx.at[b]], buf2.at[b], g2sem).wait()
            pltpu.make_async_copy(
                src_hbm.at[pl.ds(base_w, CH)], sidx.at[nb], isem).wait()
            pltpu.make_async_copy(
                dst_hbm.at[pl.ds(base_w, CH)], didx.at[nb], isem).wait()
            pltpu.async_copy(t1_hbm.at[sidx.at[nb]], buf1.at[nb], g1sem)
            pltpu.async_copy(t2_hbm.at[didx.at[nb]], buf2.at[nb], g2sem)

            @pl.when(cur >= 1)
            def _():
                scat_wait()

            didx_copy(b)
            @pl.when(cur + 2 < NCHUNK)
            def _():
                nxt = base_w + (cur + 2) * CH
                pltpu.async_copy(src_hbm.at[pl.ds(nxt, CH)], sidx.at[b], isem)
                pltpu.async_copy(dst_hbm.at[pl.ds(nxt, CH)], didx.at[b], isem)

            compute(b)
            scat_start()
        return carry

    lax.fori_loop(0, NCHUNK // 2, pair_body, 0)

    pltpu.make_async_copy(t1_hbm.at[sidx.at[0]], buf1.at[0], g1sem).wait()
    pltpu.make_async_copy(t2_hbm.at[didx.at[0]], buf2.at[0], g2sem).wait()
    scat_wait()
    didx_copy(0)
    compute(0)
    scat_start()
    scat_wait()
    plsc.subcore_barrier()

    @pl.when(jnp.logical_and(cid == 0, sid < NS - 1))
    def _():
        pltpu.sync_copy(acc.at[pl.ds(rs, RPT)], out0_hbm.at[pl.ds(rs, RPT)])

    @pl.when(jnp.logical_and(cid == 0, sid == NS - 1))
    def _():
        pltpu.sync_copy(acc.at[pl.ds((NS - 1) * RPT, RPT_LAST)],
                        out0_hbm.at[pl.ds((NS - 1) * RPT, RPT_LAST)])

    @pl.when(jnp.logical_and(cid == 1, sid < NS - 1))
    def _():
        pltpu.sync_copy(acc.at[pl.ds(rs, RPT)], out1_hbm.at[pl.ds(rs, RPT)])

    @pl.when(jnp.logical_and(cid == 1, sid == NS - 1))
    def _():
        pltpu.sync_copy(acc.at[pl.ds((NS - 1) * RPT, RPT_LAST)],
                        out1_hbm.at[pl.ds((NS - 1) * RPT, RPT_LAST)])


def _amat(a):
    ar = a.reshape(H, C).astype(jnp.float32)
    eye = jnp.eye(H, dtype=jnp.float32)
    return (ar[:, :, None] * eye[:, None, :]).reshape(D, H)


def _wfull(W, a_s, a_d):
    W = W.astype(jnp.float32)
    As = W @ _amat(a_s)
    Ad = W @ _amat(a_d)
    return jnp.concatenate([W, As, As, Ad, Ad], axis=1)


def kernel(x, edge_index, W0, as0, ad0, b0, ln0_g, ln0_b,
           W1, as1, ad1, b1, ln1_g, ln1_b,
           W2, as2, ad2, b2, ln2_g, ln2_b, linW, linb):
    src = edge_index[0].astype(jnp.int32)
    dst = edge_index[1].astype(jnp.int32)

    erep = jnp.repeat(jnp.eye(H, dtype=jnp.float32), C, axis=1)
    sfold = jnp.tile(jnp.eye(C, dtype=jnp.float32), (H, 1)) / H
    zrows = jnp.zeros((RPT_LAST, AW), jnp.float32)

    row = lambda v: v.reshape(1, -1).astype(jnp.float32)

    t1, ad16, msrc = _dense_call(x, _wfull(W0, as0, ad0))
    a0, a1 = _edge_call(t1, ad16, msrc, src, dst, zrows)

    y1, t1, ad16, msrc = _mid_call(a0, a1, x, erep, _wfull(W1, as1, ad1),
                                   row(ln0_g), row(ln0_b), row(b0))
    a0, a1 = _edge_call(t1, ad16, msrc, src, dst, zrows)

    y2, t1, ad16, msrc = _mid_call(a0, a1, y1, erep, _wfull(W2, as2, ad2),
                                   row(ln1_g), row(ln1_b), row(b1))
    a0, a1 = _edge_call(t1, ad16, msrc, src, dst, zrows)

    return _fin_call(a0, a1, erep, sfold, row(b2), row(ln2_g), row(ln2_b),
                     linW.astype(jnp.float32), row(linb))
